# baseline (speedup 1.0000x reference)
import functools

import jax
import jax.numpy as jnp
from jax import lax
from jax.experimental import pallas as pl
from jax.experimental.pallas import tpu as pltpu
from jax.experimental.pallas import tpu_sc as plsc

N = 10000
E = 320000
D = 128
HC2 = 256

NC = 2
NS = 16
NW = NC * NS
FPT = D // NW
L = 16

CH = 512
NI = 4
NCHUNK = 640
E_PAD = NCHUNK * CH
NPAD = 10240


def _sc_agg(table_flat, src_f, dst_f, w_f):
  mesh = plsc.VectorSubcoreMesh(core_axis_name="c", subcore_axis_name="s")

  @functools.partial(
      pl.kernel,
      mesh=mesh,
      compiler_params=pltpu.CompilerParams(needs_layout_passes=False),
      out_type=jax.ShapeDtypeStruct((D * NPAD,), jnp.float32),
      scratch_types=[
          pltpu.VMEM((FPT * N,), jnp.float32),
          pltpu.VMEM((FPT * NPAD,), jnp.float32),
          [pltpu.VMEM((CH,), jnp.int32) for _ in range(NI)],
          [pltpu.VMEM((CH,), jnp.int32) for _ in range(NI)],
          [pltpu.VMEM((CH,), jnp.float32) for _ in range(NI)],
          [pltpu.SemaphoreType.DMA for _ in range(NI)],
          pltpu.SemaphoreType.DMA,
      ],
  )
  def k(table_hbm, src_hbm, dst_hbm, w_hbm, out_hbm,
        tab, acc, sidx, didx, wbuf, isems, tsem):
    cid = lax.axis_index("c")
    sid = lax.axis_index("s")
    f0 = (sid * NC + cid) * FPT

    pltpu.async_copy(
        table_hbm.at[pl.ds(pl.multiple_of(f0 * N, 8), FPT * N)], tab, tsem)

    def idx_dma_start(c, s):
      sl = pl.ds(pl.multiple_of(c * CH, 8), CH)
      pltpu.async_copy(src_hbm.at[sl], sidx[s], isems[s])
      pltpu.async_copy(dst_hbm.at[sl], didx[s], isems[s])
      pltpu.async_copy(w_hbm.at[sl], wbuf[s], isems[s])

    def idx_dma_wait(c, s):
      sl = pl.ds(pl.multiple_of(c * CH, 8), CH)
      pltpu.make_async_copy(src_hbm.at[sl], sidx[s], isems[s]).wait()
      pltpu.make_async_copy(dst_hbm.at[sl], didx[s], isems[s]).wait()
      pltpu.make_async_copy(w_hbm.at[sl], wbuf[s], isems[s]).wait()

    for s in range(NI):
      idx_dma_start(s, s)

    zero = jnp.zeros((L,), jnp.float32)

    def zbody(i, carry):
      acc[pl.ds(pl.multiple_of(i * L, L), L)] = zero
      return carry

    lax.fori_loop(0, FPT * NPAD // L, zbody, 0)
    pltpu.make_async_copy(
        table_hbm.at[pl.ds(pl.multiple_of(f0 * N, 8), FPT * N)], tab,
        tsem).wait()

    def chunk_body(jj, carry):
      for u in range(NI):
        j = jj * NI + u
        idx_dma_wait(j, u)

        def grp_body(g2, c2):
          sl = pl.ds(pl.multiple_of(g2 * L, L), L)
          sv = sidx[u][sl]
          dv = didx[u][sl]
          wv = wbuf[u][sl]
          for f in range(FPT):
            gi = sv + jnp.int32(f * N)
            di = dv + jnp.int32(f * NPAD)
            vals = plsc.load_gather(tab, [gi]) * wv
            plsc.addupdate_scatter(acc, [di], vals)
          return c2

        lax.fori_loop(0, CH // L, grp_body, 0)

        @pl.when(j + NI < NCHUNK)
        def _refill():
          idx_dma_start(j + NI, u)

      return carry

    lax.fori_loop(0, NCHUNK // NI, chunk_body, 0)

    pltpu.sync_copy(
        acc, out_hbm.at[pl.ds(pl.multiple_of(f0 * NPAD, 8), FPT * NPAD)])

  out = k(table_flat, src_f, dst_f, w_f).reshape(D, NPAD)
  return out[:, :N]


def _mm(w, hT):
  return lax.dot_general(w, hT, (((1,), (0,)), ((), ())),
                         preferred_element_type=jnp.float32)


def _bn_tanh_T(hT, gamma, beta):
  mean = jnp.mean(hT, axis=1)
  c = hT - mean[:, None]
  var = jnp.mean(c * c, axis=1)
  return jnp.tanh(c * (gamma / jnp.sqrt(var + 1e-5))[:, None] + beta[:, None])


def _dense1(aggT, xT, w_rel, b, w_root, gamma, beta):
  def body(a_ref, x_ref, wrel_ref, b_ref, wroot_ref, g_ref, be_ref, o_ref):
    hT = _mm(wrel_ref[...], a_ref[...]) + b_ref[...][:, None]
    hT = hT + _mm(wroot_ref[...], x_ref[...])
    o_ref[...] = _bn_tanh_T(hT, g_ref[...], be_ref[...])

  return pl.pallas_call(
      body, out_shape=jax.ShapeDtypeStruct((D, N), jnp.float32),
  )(aggT, xT, w_rel, b, w_root, gamma, beta)


def _dense2(aggT, h1T, w_rel, b, w_root, gamma, beta, w3_rel):
  def body(a_ref, h1_ref, wrel_ref, b_ref, wroot_ref, g_ref, be_ref,
           w3_ref, h2_ref, h2t_ref):
    hT = _mm(wrel_ref[...], a_ref[...]) + b_ref[...][:, None]
    hT = hT + _mm(wroot_ref[...], h1_ref[...])
    h2T = _bn_tanh_T(hT, g_ref[...], be_ref[...])
    h2_ref[...] = h2T
    h2t_ref[...] = _mm(w3_ref[...], h2T)

  return pl.pallas_call(
      body,
      out_shape=[jax.ShapeDtypeStruct((HC2, N), jnp.float32),
                 jax.ShapeDtypeStruct((D, N), jnp.float32)],
  )(aggT, h1T, w_rel, b, w_root, gamma, beta, w3_rel)


def _dense3(aggT, h2T, b, w_root):
  def body(a_ref, h2_ref, b_ref, wroot_ref, o_ref):
    outT = a_ref[...] + b_ref[...][:, None] + _mm(wroot_ref[...], h2_ref[...])
    o_ref[...] = outT.T

  return pl.pallas_call(
      body, out_shape=jax.ShapeDtypeStruct((N, D), jnp.float32),
  )(aggT, h2T, b, w_root)


def kernel(x, edge_index, edge_attr,
           W1_rel, b1_rel, W1_root, gamma1, beta1,
           W2_rel, b2_rel, W2_root, gamma2, beta2,
           W3_rel, b3_rel, W3_root):
  pad = E_PAD - E
  src = jnp.concatenate([edge_index[0], jnp.zeros((pad,), jnp.int32)])
  dst = jnp.concatenate([edge_index[1], jnp.zeros((pad,), jnp.int32)])
  w = jnp.concatenate([edge_attr, jnp.zeros((pad,), jnp.float32)])

  xT = x.T
  a1T = _sc_agg(xT.reshape(-1), src, dst, w)
  h1T = _dense1(a1T, xT, W1_rel, b1_rel, W1_root, gamma1, beta1)
  a2T = _sc_agg(h1T.reshape(-1), src, dst, w)
  h2T, h2tT = _dense2(a2T, h1T, W2_rel, b2_rel, W2_root, gamma2, beta2,
                      W3_rel)
  a3T = _sc_agg(h2tT.reshape(-1), src, dst, w)
  return _dense3(a3T, h2T, b3_rel, W3_root)


# speedup vs baseline: 1.6966x; 1.6966x over previous
import functools

import jax
import jax.numpy as jnp
from jax import lax
from jax.experimental import pallas as pl
from jax.experimental.pallas import tpu as pltpu
from jax.experimental.pallas import tpu_sc as plsc

N = 10000
E = 320000
D = 128
DH = 64
HC2 = 256

NC = 2
NS = 16
L = 16

CH = 128
NCHUNK = 160
NBUF = 4
NI = 8
EPT = NCHUNK * CH
E_PAD = NS * EPT
NPAD = 10240
RPT = NPAD // NS
RCP = 80
NRC = RPT // RCP


def _sc_agg(table2, src_f, dst_f, w_f):
  mesh = plsc.VectorSubcoreMesh(core_axis_name="c", subcore_axis_name="s")

  @functools.partial(
      pl.kernel,
      mesh=mesh,
      compiler_params=pltpu.CompilerParams(use_tc_tiling_on_sc=False),
      out_type=jax.ShapeDtypeStruct((NC * NPAD, DH), jnp.float32),
      scratch_types=[
          [pltpu.VMEM((CH, DH), jnp.float32) for _ in range(NBUF)],
          [pltpu.VMEM((CH,), jnp.int32) for _ in range(NI)],
          [pltpu.VMEM((CH,), jnp.int32) for _ in range(NI)],
          [pltpu.VMEM((CH,), jnp.float32) for _ in range(NI)],
          pltpu.VMEM_SHARED((N, DH), jnp.float32),
          pltpu.VMEM_SHARED((NPAD, DH), jnp.float32),
          [pltpu.SemaphoreType.DMA for _ in range(NBUF)],
          [pltpu.SemaphoreType.DMA for _ in range(NBUF)],
          [pltpu.SemaphoreType.DMA for _ in range(NI)],
      ],
  )
  def k(table_hbm, src_hbm, dst_hbm, w_hbm, out_hbm,
        rows, sidx, didx, wbuf, spt, acc, gsems, ssems, isems):
    cid = lax.axis_index("c")
    sid = lax.axis_index("s")
    base = sid * EPT

    def idx_slice(ref, c):
      return ref.at[pl.ds(pl.multiple_of(base + c * CH, 8), CH)]

    def idx_dma_start(c, s):
      pltpu.async_copy(idx_slice(src_hbm, c), sidx[s], isems[s])
      pltpu.async_copy(idx_slice(dst_hbm, c), didx[s], isems[s])
      pltpu.async_copy(idx_slice(w_hbm, c), wbuf[s], isems[s])

    def idx_dma_wait(c, s):
      pltpu.make_async_copy(idx_slice(src_hbm, c), sidx[s], isems[s]).wait()
      pltpu.make_async_copy(idx_slice(dst_hbm, c), didx[s], isems[s]).wait()
      pltpu.make_async_copy(idx_slice(w_hbm, c), wbuf[s], isems[s]).wait()

    @pl.when(sid == 0)
    def _stage():
      pltpu.sync_copy(table_hbm.at[cid], spt)

    zero = jnp.zeros((L,), jnp.float32)

    def zrow(r, carry):
      for g in range(DH // L):
        rows[0][r, pl.ds(g * L, L)] = zero
      return carry

    lax.fori_loop(0, CH, zrow, 0)
    for kk in range(NRC):
      pltpu.sync_copy(rows[0].at[pl.ds(0, RCP)],
                      acc.at[pl.ds(sid * RPT + kk * RCP, RCP)])
    plsc.subcore_barrier()

    dnums = lax.GatherDimensionNumbers(
        offset_dims=(), collapsed_slice_dims=(0,), start_index_map=(0,))

    def scale(rv, wref):
      def grp_body(g2, c2):
        wv = wref[pl.ds(pl.multiple_of(g2 * L, L), L)]
        for l in range(L):
          idx = lax.broadcast(jnp.int32(l), (L,))
          wsp = lax.gather(wv, idx[:, None], dnums, (1,),
                           mode=lax.GatherScatterMode.PROMISE_IN_BOUNDS)
          e = g2 * L + l
          for g in range(DH // L):
            sl = pl.ds(g * L, L)
            rv[e, sl] = rv[e, sl] * wsp
        return c2

      lax.fori_loop(0, CH // L, grp_body, 0)

    for s in range(NI):
      idx_dma_start(s, s)
    for b in range(NBUF):
      idx_dma_wait(b, b)
      pltpu.async_copy(spt.at[sidx[b]], rows[b], gsems[b])

    def chunk_body(jj, carry):
      for u in range(NI):
        j = jj * NI + u
        b = u % NBUF
        s = u
        pltpu.make_async_copy(spt.at[sidx[s]], rows[b], gsems[b]).wait()
        scale(rows[b], wbuf[s])
        pltpu.async_copy(rows[b], acc.at[didx[s]], ssems[b], add=True)
        jn = j + NBUF - 1
        bn = (u + NBUF - 1) % NBUF
        sn = (u + NBUF - 1) % NI
        sp = (u + NI - 1) % NI

        @pl.when(jnp.logical_and(j >= 1, jn < NCHUNK))
        def _recycle():
          pltpu.make_async_copy(
              rows[bn], acc.at[didx[sp]], ssems[bn]).wait()

          @pl.when(j - 1 + NI < NCHUNK)
          def _refill():
            idx_dma_start(j - 1 + NI, sp)

          idx_dma_wait(jn, sn)
          pltpu.async_copy(spt.at[sidx[sn]], rows[bn], gsems[bn])

      return carry

    lax.fori_loop(0, NCHUNK // NI, chunk_body, 0)
    for b in range(NBUF):
      jc = NCHUNK - NBUF + b
      pltpu.make_async_copy(
          rows[b], acc.at[didx[jc % NI]], ssems[b]).wait()
    plsc.subcore_barrier()

    for kk in range(NRC):
      row0 = sid * RPT + kk * RCP
      pltpu.sync_copy(acc.at[pl.ds(row0, RCP)],
                      out_hbm.at[pl.ds(cid * NPAD + row0, RCP)])

  out = k(table2, src_f, dst_f, w_f).reshape(NC, NPAD, DH)
  return out[0, :N, :], out[1, :N, :]


def _dotT(a, w):
  return lax.dot_general(a, w, (((1,), (1,)), ((), ())),
                         preferred_element_type=jnp.float32)


def _agg_dot(p_lo, p_hi, w_rel):
  return _dotT(p_lo, w_rel[:, :DH]) + _dotT(p_hi, w_rel[:, DH:])


def _bn_tanh(h, gamma, beta):
  mean = jnp.mean(h, axis=0)
  c = h - mean[None, :]
  var = jnp.mean(c * c, axis=0)
  return jnp.tanh(c * (gamma / jnp.sqrt(var + 1e-5))[None, :] + beta[None, :])


def _split(h):
  return jnp.stack([h[:, :DH], h[:, DH:]])


def _dense1(p_lo, p_hi, x, w_rel, b, w_root, gamma, beta):
  def body(pl_ref, ph_ref, x_ref, wrel_ref, b_ref, wroot_ref, g_ref, be_ref,
           o_ref):
    h = _agg_dot(pl_ref[...], ph_ref[...], wrel_ref[...]) + b_ref[...][None, :]
    h = h + _dotT(x_ref[...], wroot_ref[...])
    o_ref[...] = _split(_bn_tanh(h, g_ref[...], be_ref[...]))

  return pl.pallas_call(
      body, out_shape=jax.ShapeDtypeStruct((NC, N, DH), jnp.float32),
  )(p_lo, p_hi, x, w_rel, b, w_root, gamma, beta)


def _dense2(p_lo, p_hi, h1s, w_rel, b, w_root, gamma, beta, w3_rel):
  def body(pl_ref, ph_ref, h1_ref, wrel_ref, b_ref, wroot_ref, g_ref, be_ref,
           w3_ref, h2_ref, h2t_ref):
    h1 = jnp.concatenate([h1_ref[0], h1_ref[1]], axis=1)
    h = _agg_dot(pl_ref[...], ph_ref[...], wrel_ref[...]) + b_ref[...][None, :]
    h = h + _dotT(h1, wroot_ref[...])
    h2 = _bn_tanh(h, g_ref[...], be_ref[...])
    h2_ref[...] = h2
    h2t_ref[...] = _split(_dotT(h2, w3_ref[...]))

  return pl.pallas_call(
      body,
      out_shape=[jax.ShapeDtypeStruct((N, HC2), jnp.float32),
                 jax.ShapeDtypeStruct((NC, N, DH), jnp.float32)],
  )(p_lo, p_hi, h1s, w_rel, b, w_root, gamma, beta, w3_rel)


def _dense3(p_lo, p_hi, h2, b, w_root):
  def body(pl_ref, ph_ref, h2_ref, b_ref, wroot_ref, o_ref):
    agg = jnp.concatenate([pl_ref[...], ph_ref[...]], axis=1)
    o_ref[...] = agg + b_ref[...][None, :] + _dotT(h2_ref[...], wroot_ref[...])

  return pl.pallas_call(
      body, out_shape=jax.ShapeDtypeStruct((N, D), jnp.float32),
  )(p_lo, p_hi, h2, b, w_root)


def kernel(x, edge_index, edge_attr,
           W1_rel, b1_rel, W1_root, gamma1, beta1,
           W2_rel, b2_rel, W2_root, gamma2, beta2,
           W3_rel, b3_rel, W3_root):
  pad = E_PAD - E
  src = jnp.concatenate([edge_index[0], jnp.zeros((pad,), jnp.int32)])
  dst = jnp.concatenate([edge_index[1], jnp.zeros((pad,), jnp.int32)])
  w = jnp.concatenate([edge_attr, jnp.zeros((pad,), jnp.float32)])

  xs = jnp.stack([x[:, :DH], x[:, DH:]])
  a1_lo, a1_hi = _sc_agg(xs, src, dst, w)
  h1s = _dense1(a1_lo, a1_hi, x, W1_rel, b1_rel, W1_root, gamma1, beta1)
  a2_lo, a2_hi = _sc_agg(h1s, src, dst, w)
  h2, h2ts = _dense2(a2_lo, a2_hi, h1s, W2_rel, b2_rel, W2_root, gamma2,
                     beta2, W3_rel)
  a3_lo, a3_hi = _sc_agg(h2ts, src, dst, w)
  return _dense3(a3_lo, a3_hi, h2, b3_rel, W3_root)


# speedup vs baseline: 1.7256x; 1.0171x over previous
import functools

import jax
import jax.numpy as jnp
from jax import lax
from jax.experimental import pallas as pl
from jax.experimental.pallas import tpu as pltpu
from jax.experimental.pallas import tpu_sc as plsc

N = 10000
E = 320000
D = 128
DH = 64
HC2 = 256

NC = 2
NS = 16
L = 16

CH = 128
NCHUNK = 160
NBUF = 4
NI = 8
EPT = NCHUNK * CH
E_PAD = NS * EPT
NPAD = 10240
RPT = NPAD // NS
RCP = 80
NRC = RPT // RCP


def _sc_agg(table2, src_f, dst_f, w_f):
  mesh = plsc.VectorSubcoreMesh(core_axis_name="c", subcore_axis_name="s")

  @functools.partial(
      pl.kernel,
      mesh=mesh,
      compiler_params=pltpu.CompilerParams(use_tc_tiling_on_sc=False),
      out_type=jax.ShapeDtypeStruct((NC * NPAD, DH), jnp.float32),
      scratch_types=[
          [pltpu.VMEM((CH, DH), jnp.float32) for _ in range(NBUF)],
          [pltpu.VMEM((CH,), jnp.int32) for _ in range(NI)],
          [pltpu.VMEM((CH,), jnp.int32) for _ in range(NI)],
          [pltpu.VMEM((CH,), jnp.float32) for _ in range(NI)],
          pltpu.VMEM_SHARED((N, DH), jnp.float32),
          pltpu.VMEM_SHARED((NPAD, DH), jnp.float32),
          [pltpu.SemaphoreType.DMA for _ in range(NBUF)],
          [pltpu.SemaphoreType.DMA for _ in range(NBUF)],
          [pltpu.SemaphoreType.DMA for _ in range(NI)],
      ],
  )
  def k(table_hbm, src_hbm, dst_hbm, w_hbm, out_hbm,
        rows, sidx, didx, wbuf, spt, acc, gsems, ssems, isems):
    cid = lax.axis_index("c")
    sid = lax.axis_index("s")
    base = sid * EPT

    def idx_slice(ref, c):
      return ref.at[pl.ds(pl.multiple_of(base + c * CH, 8), CH)]

    def idx_dma_start(c, s):
      pltpu.async_copy(idx_slice(src_hbm, c), sidx[s], isems[s])
      pltpu.async_copy(idx_slice(dst_hbm, c), didx[s], isems[s])
      pltpu.async_copy(idx_slice(w_hbm, c), wbuf[s], isems[s])

    def idx_dma_wait(c, s):
      pltpu.make_async_copy(idx_slice(src_hbm, c), sidx[s], isems[s]).wait()
      pltpu.make_async_copy(idx_slice(dst_hbm, c), didx[s], isems[s]).wait()
      pltpu.make_async_copy(idx_slice(w_hbm, c), wbuf[s], isems[s]).wait()

    @pl.when(sid == 0)
    def _stage():
      pltpu.sync_copy(table_hbm.at[cid], spt)

    zero = jnp.zeros((L,), jnp.float32)

    def zrow(r, carry):
      for g in range(DH // L):
        rows[0][r, pl.ds(g * L, L)] = zero
      return carry

    lax.fori_loop(0, CH, zrow, 0)
    for kk in range(NRC):
      pltpu.sync_copy(rows[0].at[pl.ds(0, RCP)],
                      acc.at[pl.ds(sid * RPT + kk * RCP, RCP)])
    plsc.subcore_barrier()

    dnums = lax.GatherDimensionNumbers(
        offset_dims=(), collapsed_slice_dims=(0,), start_index_map=(0,))

    def scale(rv, wref):
      def grp_body(g2, c2):
        wv = wref[pl.ds(pl.multiple_of(g2 * L, L), L)]
        for l in range(L):
          idx = lax.broadcast(jnp.int32(l), (L,))
          wsp = lax.gather(wv, idx[:, None], dnums, (1,),
                           mode=lax.GatherScatterMode.PROMISE_IN_BOUNDS)
          e = g2 * L + l
          for g in range(DH // L):
            sl = pl.ds(g * L, L)
            rv[e, sl] = rv[e, sl] * wsp
        return c2

      lax.fori_loop(0, CH // L, grp_body, 0)

    for s in range(NI):
      idx_dma_start(s, s)
    for b in range(NBUF):
      idx_dma_wait(b, b)
      pltpu.async_copy(spt.at[sidx[b]], rows[b], gsems[b])

    def chunk_body(jj, carry):
      for u in range(NI):
        j = jj * NI + u
        b = u % NBUF
        s = u
        pltpu.make_async_copy(spt.at[sidx[s]], rows[b], gsems[b]).wait()
        scale(rows[b], wbuf[s])
        pltpu.async_copy(rows[b], acc.at[didx[s]], ssems[b], add=True)
        jn = j + NBUF - 1
        bn = (u + NBUF - 1) % NBUF
        sn = (u + NBUF - 1) % NI
        sp = (u + NI - 1) % NI

        @pl.when(jnp.logical_and(j >= 1, jn < NCHUNK))
        def _recycle():
          pltpu.make_async_copy(
              rows[bn], acc.at[didx[sp]], ssems[bn]).wait()

          @pl.when(j - 1 + NI < NCHUNK)
          def _refill():
            idx_dma_start(j - 1 + NI, sp)

          idx_dma_wait(jn, sn)
          pltpu.async_copy(spt.at[sidx[sn]], rows[bn], gsems[bn])

      return carry

    lax.fori_loop(0, NCHUNK // NI, chunk_body, 0)
    for b in range(NBUF):
      jc = NCHUNK - NBUF + b
      pltpu.make_async_copy(
          rows[b], acc.at[didx[jc % NI]], ssems[b]).wait()
    plsc.subcore_barrier()

    for kk in range(NRC):
      row0 = sid * RPT + kk * RCP
      pltpu.sync_copy(acc.at[pl.ds(row0, RCP)],
                      out_hbm.at[pl.ds(cid * NPAD + row0, RCP)])

  return k(table2, src_f, dst_f, w_f)


def _halves(a_ref):
  return a_ref[pl.ds(0, N), :], a_ref[pl.ds(NPAD, N), :]


def _dotT(a, w):
  return lax.dot_general(a, w, (((1,), (1,)), ((), ())),
                         preferred_element_type=jnp.float32)


def _agg_dot(p_lo, p_hi, w_rel):
  return _dotT(p_lo, w_rel[:, :DH]) + _dotT(p_hi, w_rel[:, DH:])


def _bn_tanh(h, gamma, beta):
  mean = jnp.mean(h, axis=0)
  c = h - mean[None, :]
  var = jnp.mean(c * c, axis=0)
  return jnp.tanh(c * (gamma / jnp.sqrt(var + 1e-5))[None, :] + beta[None, :])


def _split(h):
  return jnp.stack([h[:, :DH], h[:, DH:]])


def _dense1(agg_raw, x, w_rel, b, w_root, gamma, beta):
  def body(a_ref, x_ref, wrel_ref, b_ref, wroot_ref, g_ref, be_ref, o_ref):
    p_lo, p_hi = _halves(a_ref)
    h = _agg_dot(p_lo, p_hi, wrel_ref[...]) + b_ref[...][None, :]
    h = h + _dotT(x_ref[...], wroot_ref[...])
    o_ref[...] = _split(_bn_tanh(h, g_ref[...], be_ref[...]))

  return pl.pallas_call(
      body, out_shape=jax.ShapeDtypeStruct((NC, N, DH), jnp.float32),
  )(agg_raw, x, w_rel, b, w_root, gamma, beta)


def _dense2(agg_raw, h1s, w_rel, b, w_root, gamma, beta, w3_rel):
  def body(a_ref, h1_ref, wrel_ref, b_ref, wroot_ref, g_ref, be_ref,
           w3_ref, h2_ref, h2t_ref):
    p_lo, p_hi = _halves(a_ref)
    h1 = jnp.concatenate([h1_ref[0], h1_ref[1]], axis=1)
    h = _agg_dot(p_lo, p_hi, wrel_ref[...]) + b_ref[...][None, :]
    h = h + _dotT(h1, wroot_ref[...])
    h2 = _bn_tanh(h, g_ref[...], be_ref[...])
    h2_ref[...] = h2
    h2t_ref[...] = _split(_dotT(h2, w3_ref[...]))

  return pl.pallas_call(
      body,
      out_shape=[jax.ShapeDtypeStruct((N, HC2), jnp.float32),
                 jax.ShapeDtypeStruct((NC, N, DH), jnp.float32)],
  )(agg_raw, h1s, w_rel, b, w_root, gamma, beta, w3_rel)


def _dense3(agg_raw, h2, b, w_root):
  def body(a_ref, h2_ref, b_ref, wroot_ref, o_ref):
    p_lo, p_hi = _halves(a_ref)
    agg = jnp.concatenate([p_lo, p_hi], axis=1)
    o_ref[...] = agg + b_ref[...][None, :] + _dotT(h2_ref[...], wroot_ref[...])

  return pl.pallas_call(
      body, out_shape=jax.ShapeDtypeStruct((N, D), jnp.float32),
  )(agg_raw, h2, b, w_root)


def kernel(x, edge_index, edge_attr,
           W1_rel, b1_rel, W1_root, gamma1, beta1,
           W2_rel, b2_rel, W2_root, gamma2, beta2,
           W3_rel, b3_rel, W3_root):
  pad = E_PAD - E
  src = jnp.concatenate([edge_index[0], jnp.zeros((pad,), jnp.int32)])
  dst = jnp.concatenate([edge_index[1], jnp.zeros((pad,), jnp.int32)])
  w = jnp.concatenate([edge_attr, jnp.zeros((pad,), jnp.float32)])

  xs = jnp.stack([x[:, :DH], x[:, DH:]])
  a1 = _sc_agg(xs, src, dst, w)
  h1s = _dense1(a1, x, W1_rel, b1_rel, W1_root, gamma1, beta1)
  a2 = _sc_agg(h1s, src, dst, w)
  h2, h2ts = _dense2(a2, h1s, W2_rel, b2_rel, W2_root, gamma2, beta2, W3_rel)
  a3 = _sc_agg(h2ts, src, dst, w)
  return _dense3(a3, h2, b3_rel, W3_root)
